# baseline (speedup 1.0000x reference)
import functools

import jax
import jax.numpy as jnp
from jax.experimental import pallas as pl
from jax.experimental.pallas import tpu as pltpu

N = 400
HID = 64
H = 4
OUT = 32
_F32 = jnp.float32


def _bdot(a, b):
    return jax.lax.dot_general(
        a.astype(jnp.bfloat16), b.astype(jnp.bfloat16),
        (((1,), (0,)), ((), ())), preferred_element_type=_F32)


def _layer_norm_rows(x, g, b, eps=1e-5):
    mu = x.mean(axis=1, keepdims=True)
    var = ((x - mu) ** 2).mean(axis=1, keepdims=True)
    return (x - mu) / jnp.sqrt(var + eps) * g + b


def _gat(x, mask_t, w, att_src, att_dst, bias):
    xh = _bdot(x, w)
    ones_col = jnp.ones((N, 1), _F32)
    acc = jnp.zeros((N, HID), _F32)
    for h in range(H):
        xh_h = xh[:, h * HID:(h + 1) * HID]
        adst_col = (xh_h * att_dst[h:h + 1, :]).sum(axis=1, keepdims=True)
        asrc_row = jax.lax.dot_general(
            att_src[h:h + 1, :], xh_h, (((1,), (1,)), ((), ())),
            preferred_element_type=_F32)
        e = asrc_row + adst_col
        e = jnp.maximum(e, 0.2 * e)
        e = jnp.where(mask_t, e, -jnp.inf)
        mb = jnp.max(asrc_row) + adst_col
        mb = jnp.maximum(mb, 0.2 * mb)
        ex = jnp.exp(e - mb)
        den = _bdot(ex, ones_col)
        acc = acc + _bdot(ex, xh_h) / den
    return acc * (1.0 / H) + bias


def _pool(v, gate_row):
    s = (v * gate_row).sum(axis=1, keepdims=True)
    p = jnp.exp(s - jnp.max(s))
    p = p / jnp.sum(p)
    return (p * v).sum(axis=0, keepdims=True)


def _branch(mode, mat_ref, diag, gmp_w_ref, gmp_b_ref, w1_ref, ln1_g_ref,
            ln1_b_ref, w2_ref, gat_refs, gln_g_ref, gln_b_ref, gate_row_ref,
            ow1_ref, ob1_ref, ow2_ref, ob2_ref, out_ref):
    m = mat_ref[0]
    m3 = m * m * m

    ones_col = jnp.ones((N, 1), _F32)
    a = jnp.abs(m3)
    rs = jax.lax.dot_general(a, ones_col, (((1,), (0,)), ((), ())),
                             preferred_element_type=_F32)
    mu_s = jnp.sum(rs) / N
    sd_s = jnp.sqrt(jnp.sum((rs - mu_s) ** 2) / (N - 1)) + 1e-6
    strength = (rs - mu_s) / sd_s
    rsp = rs + 1e-6
    al = jax.lax.dot_general(a * jnp.log(a + 1e-9 * rsp), ones_col,
                             (((1,), (0,)), ((), ())),
                             preferred_element_type=_F32)
    ent = (jnp.log(rsp) * rs - al) / rsp
    mu_e = jnp.sum(ent) / N
    sd_e = jnp.sqrt(jnp.sum((ent - mu_e) ** 2) / (N - 1)) + 1e-6
    ent = (ent - mu_e) / sd_e
    me = (strength * gmp_w_ref[0:1, :] + ent * gmp_w_ref[1:2, :]
          + gmp_b_ref[...])

    aug = 100.0 * m3 + jnp.where(diag, 1.0, 0.0)
    h = _bdot(aug, w1_ref[0:N, :]) + _bdot(me, w1_ref[N:, :])
    h = _layer_norm_rows(h, ln1_g_ref[...], ln1_b_ref[...])
    h = jnp.maximum(h, 0.0)
    h = _bdot(h, w2_ref[...])
    mu_c = h.mean(axis=0, keepdims=True)
    var_c = ((h - mu_c) ** 2).mean(axis=0, keepdims=True)
    x = (h - mu_c) / jnp.sqrt(var_c + 1e-5)

    m3t = m3.T
    if mode == 'dti':
        mask_t = (m3t != 0.0) | diag
        xg = _gat(x, mask_t, *gat_refs[0])
        xg = _layer_norm_rows(xg, gln_g_ref[...], gln_b_ref[...])
    else:
        xp = _gat(x, (m3t > 0.0) | diag, *gat_refs[0])
        xn = _gat(x, (m3t < 0.0) | diag, *gat_refs[1])
        xg = _layer_norm_rows(xp - xn, gln_g_ref[...], gln_b_ref[...])
    xg = jnp.maximum(xg, 0.0) + x

    gate_row = gate_row_ref[...]
    pg = _pool(xg, gate_row)
    pr = _pool(x, gate_row)
    z = (_bdot(pg, ow1_ref[0:HID, :]) + _bdot(pr * 2.0, ow1_ref[HID:, :])
         + ob1_ref[...])
    z = jnp.maximum(z, 0.0)
    z = _bdot(z, ow2_ref[...]) + ob2_ref[...]
    out_ref[0] = z * (1.0 / jnp.sqrt(jnp.float32(1.0 + 1e-5)))


def _unpack(refs, idx, n_gat):
    head = refs[idx:idx + 6]
    idx += 6
    gat_refs = []
    for _ in range(n_gat):
        gat_refs.append(tuple(r[...] for r in refs[idx:idx + 4]))
        idx += 4
    tail = refs[idx:idx + 7]
    return head, gat_refs, tail, idx + 7


def _body(*refs):
    dti_mat, fmri_mat = refs[0], refs[1]
    d_head, d_gats, d_tail, idx = _unpack(refs, 2, 1)
    f_head, f_gats, f_tail, idx = _unpack(refs, idx, 2)
    out_d, out_f = refs[idx], refs[idx + 1]
    ii = jax.lax.broadcasted_iota(jnp.int32, (N, N), 0)
    jj = jax.lax.broadcasted_iota(jnp.int32, (N, N), 1)
    diag = ii == jj
    _branch('dti', dti_mat, diag, *d_head, d_gats, *d_tail, out_d)
    _branch('fmri', fmri_mat, diag, *f_head, f_gats, *f_tail, out_f)


def _branch_weights(p, pre, gat_names):
    row = lambda v: v.reshape(1, -1)
    ws = [p['gmp_w'], row(p['gmp_b']), p[pre + '_mlp_w1'],
          row(p[pre + '_ln1_g']), row(p[pre + '_ln1_b']), p[pre + '_mlp_w2']]
    for gn in gat_names:
        g = p[gn]
        ws += [g['w'], g['att_src'], g['att_dst'], row(g['bias'])]
    ws += [row(p[pre + '_gln_g']), row(p[pre + '_gln_b']),
           p[pre + '_gate_w'].reshape(1, -1), p[pre + '_out_w1'],
           row(p[pre + '_out_b1']), p[pre + '_out_w2'],
           row(p[pre + '_out_b2'])]
    return ws


def kernel(dti_matrix, fmri_matrix, params):
    B = dti_matrix.shape[0]
    weights = (_branch_weights(params, 'dti', ['dti_gat'])
               + _branch_weights(params, 'fmri',
                                 ['fmri_gat_pos', 'fmri_gat_neg']))
    in_specs = [pl.BlockSpec((1, N, N), lambda b: (b, 0, 0)),
                pl.BlockSpec((1, N, N), lambda b: (b, 0, 0))]
    for w in weights:
        in_specs.append(
            pl.BlockSpec(w.shape, functools.partial(
                lambda b, _r: (0,) * _r, _r=w.ndim)))
    out_spec = pl.BlockSpec((1, 1, OUT), lambda b: (b, 0, 0))
    d, f = pl.pallas_call(
        _body,
        grid=(B,),
        in_specs=in_specs,
        out_specs=[out_spec, out_spec],
        out_shape=[jax.ShapeDtypeStruct((B, 1, OUT), _F32)] * 2,
        compiler_params=pltpu.CompilerParams(
            dimension_semantics=("parallel",)),
    )(dti_matrix, fmri_matrix, *weights)
    return d.reshape(B, OUT), f.reshape(B, OUT)


# speedup vs baseline: 1.0489x; 1.0489x over previous
import functools

import jax
import jax.numpy as jnp
from jax.experimental import pallas as pl
from jax.experimental.pallas import tpu as pltpu

N = 400
HID = 64
H = 4
OUT = 32
_F32 = jnp.float32


def _bdot(a, b):
    return jax.lax.dot_general(
        a.astype(jnp.bfloat16), b.astype(jnp.bfloat16),
        (((1,), (0,)), ((), ())), preferred_element_type=_F32)


def _layer_norm_rows(x, g, b, eps=1e-5):
    mu = x.mean(axis=1, keepdims=True)
    var = ((x - mu) ** 2).mean(axis=1, keepdims=True)
    return (x - mu) / jnp.sqrt(var + eps) * g + b


def _gat(x, mask_t, w, att_src, att_dst, bias):
    xh = _bdot(x, w)
    ones_col = jnp.ones((N, 1), _F32)
    acc = jnp.zeros((N, HID), _F32)
    for h in range(H):
        xh_h = xh[:, h * HID:(h + 1) * HID]
        adst_col = (xh_h * att_dst[h:h + 1, :]).sum(axis=1, keepdims=True)
        asrc_row = jax.lax.dot_general(
            att_src[h:h + 1, :], xh_h, (((1,), (1,)), ((), ())),
            preferred_element_type=_F32)
        e = asrc_row + adst_col
        e = jnp.maximum(e, 0.2 * e)
        e = jnp.where(mask_t, e, -jnp.inf)
        m = jnp.max(e, axis=1, keepdims=True)
        ex = jnp.exp(e - m)
        acc = acc + _bdot(ex, xh_h) / ex.sum(axis=1, keepdims=True)
    return acc * (1.0 / H) + bias


def _pool(v, gate_row):
    s = (v * gate_row).sum(axis=1, keepdims=True)
    p = jnp.exp(s - jnp.max(s))
    p = p / jnp.sum(p)
    return (p * v).sum(axis=0, keepdims=True)


def _branch(mode, mat_ref, diag, gmp_w_ref, gmp_b_ref, w1_ref, ln1_g_ref,
            ln1_b_ref, w2_ref, gat_refs, gln_g_ref, gln_b_ref, gate_row_ref,
            ow1_ref, ob1_ref, ow2_ref, ob2_ref, out_ref):
    m = mat_ref[0]
    m3 = m * m * m

    ones_col = jnp.ones((N, 1), _F32)
    a = jnp.abs(m3)
    rs = jax.lax.dot_general(a, ones_col, (((1,), (0,)), ((), ())),
                             preferred_element_type=_F32)
    mu_s = jnp.sum(rs) / N
    sd_s = jnp.sqrt(jnp.sum((rs - mu_s) ** 2) / (N - 1)) + 1e-6
    strength = (rs - mu_s) / sd_s
    rsp = rs + 1e-6
    al = jax.lax.dot_general(a * jnp.log(a + 1e-9 * rsp), ones_col,
                             (((1,), (0,)), ((), ())),
                             preferred_element_type=_F32)
    ent = (jnp.log(rsp) * rs - al) / rsp
    mu_e = jnp.sum(ent) / N
    sd_e = jnp.sqrt(jnp.sum((ent - mu_e) ** 2) / (N - 1)) + 1e-6
    ent = (ent - mu_e) / sd_e
    me = (strength * gmp_w_ref[0:1, :] + ent * gmp_w_ref[1:2, :]
          + gmp_b_ref[...])

    aug = 100.0 * m3 + jnp.where(diag, 1.0, 0.0)
    h = _bdot(aug, w1_ref[0:N, :]) + _bdot(me, w1_ref[N:, :])
    h = _layer_norm_rows(h, ln1_g_ref[...], ln1_b_ref[...])
    h = jnp.maximum(h, 0.0)
    h = _bdot(h, w2_ref[...])
    mu_c = h.mean(axis=0, keepdims=True)
    var_c = ((h - mu_c) ** 2).mean(axis=0, keepdims=True)
    x = (h - mu_c) / jnp.sqrt(var_c + 1e-5)

    m3t = m3.T
    if mode == 'dti':
        mask_t = (m3t != 0.0) | diag
        xg = _gat(x, mask_t, *gat_refs[0])
        xg = _layer_norm_rows(xg, gln_g_ref[...], gln_b_ref[...])
    else:
        xp = _gat(x, (m3t > 0.0) | diag, *gat_refs[0])
        xn = _gat(x, (m3t < 0.0) | diag, *gat_refs[1])
        xg = _layer_norm_rows(xp - xn, gln_g_ref[...], gln_b_ref[...])
    xg = jnp.maximum(xg, 0.0) + x

    gate_row = gate_row_ref[...]
    pg = _pool(xg, gate_row)
    pr = _pool(x, gate_row)
    z = (_bdot(pg, ow1_ref[0:HID, :]) + _bdot(pr * 2.0, ow1_ref[HID:, :])
         + ob1_ref[...])
    z = jnp.maximum(z, 0.0)
    z = _bdot(z, ow2_ref[...]) + ob2_ref[...]
    out_ref[0] = z * (1.0 / jnp.sqrt(jnp.float32(1.0 + 1e-5)))


def _unpack(refs, idx, n_gat):
    head = refs[idx:idx + 6]
    idx += 6
    gat_refs = []
    for _ in range(n_gat):
        gat_refs.append(tuple(r[...] for r in refs[idx:idx + 4]))
        idx += 4
    tail = refs[idx:idx + 7]
    return head, gat_refs, tail, idx + 7


def _body(*refs):
    dti_mat, fmri_mat = refs[0], refs[1]
    d_head, d_gats, d_tail, idx = _unpack(refs, 2, 1)
    f_head, f_gats, f_tail, idx = _unpack(refs, idx, 2)
    out_d, out_f = refs[idx], refs[idx + 1]
    ii = jax.lax.broadcasted_iota(jnp.int32, (N, N), 0)
    jj = jax.lax.broadcasted_iota(jnp.int32, (N, N), 1)
    diag = ii == jj
    _branch('dti', dti_mat, diag, *d_head, d_gats, *d_tail, out_d)
    _branch('fmri', fmri_mat, diag, *f_head, f_gats, *f_tail, out_f)


def _branch_weights(p, pre, gat_names):
    row = lambda v: v.reshape(1, -1)
    ws = [p['gmp_w'], row(p['gmp_b']), p[pre + '_mlp_w1'],
          row(p[pre + '_ln1_g']), row(p[pre + '_ln1_b']), p[pre + '_mlp_w2']]
    for gn in gat_names:
        g = p[gn]
        ws += [g['w'], g['att_src'], g['att_dst'], row(g['bias'])]
    ws += [row(p[pre + '_gln_g']), row(p[pre + '_gln_b']),
           p[pre + '_gate_w'].reshape(1, -1), p[pre + '_out_w1'],
           row(p[pre + '_out_b1']), p[pre + '_out_w2'],
           row(p[pre + '_out_b2'])]
    return ws


def kernel(dti_matrix, fmri_matrix, params):
    B = dti_matrix.shape[0]
    weights = (_branch_weights(params, 'dti', ['dti_gat'])
               + _branch_weights(params, 'fmri',
                                 ['fmri_gat_pos', 'fmri_gat_neg']))
    in_specs = [pl.BlockSpec((1, N, N), lambda b: (b, 0, 0)),
                pl.BlockSpec((1, N, N), lambda b: (b, 0, 0))]
    for w in weights:
        in_specs.append(
            pl.BlockSpec(w.shape, functools.partial(
                lambda b, _r: (0,) * _r, _r=w.ndim)))
    out_spec = pl.BlockSpec((1, 1, OUT), lambda b: (b, 0, 0))
    d, f = pl.pallas_call(
        _body,
        grid=(B,),
        in_specs=in_specs,
        out_specs=[out_spec, out_spec],
        out_shape=[jax.ShapeDtypeStruct((B, 1, OUT), _F32)] * 2,
        compiler_params=pltpu.CompilerParams(
            dimension_semantics=("parallel",)),
    )(dti_matrix, fmri_matrix, *weights)
    return d.reshape(B, OUT), f.reshape(B, OUT)


# speedup vs baseline: 1.1835x; 1.1284x over previous
import functools

import jax
import jax.numpy as jnp
from jax.experimental import pallas as pl
from jax.experimental.pallas import tpu as pltpu

N = 400
HID = 64
H = 4
OUT = 32
_F32 = jnp.float32


def _bdot(a, b):
    return jax.lax.dot_general(
        a.astype(jnp.bfloat16), b.astype(jnp.bfloat16),
        (((1,), (0,)), ((), ())), preferred_element_type=_F32)


def _layer_norm_rows(x, g, b, eps=1e-5):
    mu = x.mean(axis=1, keepdims=True)
    var = ((x - mu) ** 2).mean(axis=1, keepdims=True)
    return (x - mu) / jnp.sqrt(var + eps) * g + b


def _gat(x, mask_t, w, att_src, att_dst, bias):
    xh = _bdot(x, w)
    acc = jnp.zeros((N, HID), _F32)
    for h in range(H):
        xh_h = xh[:, h * HID:(h + 1) * HID]
        adst_col = (xh_h * att_dst[h:h + 1, :]).sum(axis=1, keepdims=True)
        asrc_row = jax.lax.dot_general(
            att_src[h:h + 1, :], xh_h, (((1,), (1,)), ((), ())),
            preferred_element_type=_F32)
        e = asrc_row + adst_col
        e = jnp.maximum(e, 0.2 * e)
        e = jnp.where(mask_t, e, -jnp.inf)
        m = jnp.max(e, axis=1, keepdims=True)
        ex = jnp.exp(e - m)
        acc = acc + _bdot(ex, xh_h) / ex.sum(axis=1, keepdims=True)
    return acc * (1.0 / H) + bias


def _pool(v, gate_row):
    s = (v * gate_row).sum(axis=1, keepdims=True)
    p = jnp.exp(s - jnp.max(s))
    p = p / jnp.sum(p)
    return (p * v).sum(axis=0, keepdims=True)


def _branch(mode, mat_ref, diag, gmp_w_ref, gmp_b_ref, w1_ref, ln1_g_ref,
            ln1_b_ref, w2_ref, gat_refs, gln_g_ref, gln_b_ref, gate_row_ref,
            ow1_ref, ob1_ref, ow2_ref, ob2_ref, out_ref):
    m = mat_ref[0]
    m3 = m * m * m

    a = jnp.abs(m3)
    rs = a.sum(axis=1, keepdims=True)
    mu_s = jnp.sum(rs) / N
    sd_s = jnp.sqrt(jnp.sum((rs - mu_s) ** 2) / (N - 1)) + 1e-6
    strength = (rs - mu_s) / sd_s
    rsp = rs + 1e-6
    al = (a * jnp.log(a + 1e-9 * rsp)).sum(axis=1, keepdims=True)
    ent = (jnp.log(rsp) * rs - al) / rsp
    mu_e = jnp.sum(ent) / N
    sd_e = jnp.sqrt(jnp.sum((ent - mu_e) ** 2) / (N - 1)) + 1e-6
    ent = (ent - mu_e) / sd_e
    me = (strength * gmp_w_ref[0:1, :] + ent * gmp_w_ref[1:2, :]
          + gmp_b_ref[...])

    aug = 100.0 * m3 + jnp.where(diag, 1.0, 0.0)
    h = _bdot(aug, w1_ref[0:N, :]) + _bdot(me, w1_ref[N:, :])
    h = _layer_norm_rows(h, ln1_g_ref[...], ln1_b_ref[...])
    h = jnp.maximum(h, 0.0)
    h = _bdot(h, w2_ref[...])
    mu_c = h.mean(axis=0, keepdims=True)
    var_c = ((h - mu_c) ** 2).mean(axis=0, keepdims=True)
    x = (h - mu_c) / jnp.sqrt(var_c + 1e-5)

    m3t = m3.T
    if mode == 'dti':
        mask_t = (m3t != 0.0) | diag
        xg = _gat(x, mask_t, *gat_refs[0])
        xg = _layer_norm_rows(xg, gln_g_ref[...], gln_b_ref[...])
    else:
        xp = _gat(x, (m3t > 0.0) | diag, *gat_refs[0])
        xn = _gat(x, (m3t < 0.0) | diag, *gat_refs[1])
        xg = _layer_norm_rows(xp - xn, gln_g_ref[...], gln_b_ref[...])
    xg = jnp.maximum(xg, 0.0) + x

    gate_row = gate_row_ref[...]
    pg = _pool(xg, gate_row)
    pr = _pool(x, gate_row)
    z = (_bdot(pg, ow1_ref[0:HID, :]) + _bdot(pr * 2.0, ow1_ref[HID:, :])
         + ob1_ref[...])
    z = jnp.maximum(z, 0.0)
    z = _bdot(z, ow2_ref[...]) + ob2_ref[...]
    out_ref[0] = z * (1.0 / jnp.sqrt(jnp.float32(1.0 + 1e-5)))


def _unpack(refs, idx, n_gat):
    head = refs[idx:idx + 6]
    idx += 6
    gat_refs = []
    for _ in range(n_gat):
        gat_refs.append(tuple(r[...] for r in refs[idx:idx + 4]))
        idx += 4
    tail = refs[idx:idx + 7]
    return head, gat_refs, tail, idx + 7


def _body(*refs):
    dti_mat, fmri_mat = refs[0], refs[1]
    d_head, d_gats, d_tail, idx = _unpack(refs, 2, 1)
    f_head, f_gats, f_tail, idx = _unpack(refs, idx, 2)
    out_d, out_f = refs[idx], refs[idx + 1]
    ii = jax.lax.broadcasted_iota(jnp.int32, (N, N), 0)
    jj = jax.lax.broadcasted_iota(jnp.int32, (N, N), 1)
    diag = ii == jj
    _branch('dti', dti_mat, diag, *d_head, d_gats, *d_tail, out_d)
    _branch('fmri', fmri_mat, diag, *f_head, f_gats, *f_tail, out_f)


def _branch_weights(p, pre, gat_names):
    row = lambda v: v.reshape(1, -1)
    ws = [p['gmp_w'], row(p['gmp_b']), p[pre + '_mlp_w1'],
          row(p[pre + '_ln1_g']), row(p[pre + '_ln1_b']), p[pre + '_mlp_w2']]
    for gn in gat_names:
        g = p[gn]
        ws += [g['w'], g['att_src'], g['att_dst'], row(g['bias'])]
    ws += [row(p[pre + '_gln_g']), row(p[pre + '_gln_b']),
           p[pre + '_gate_w'].reshape(1, -1), p[pre + '_out_w1'],
           row(p[pre + '_out_b1']), p[pre + '_out_w2'],
           row(p[pre + '_out_b2'])]
    return ws


def kernel(dti_matrix, fmri_matrix, params):
    B = dti_matrix.shape[0]
    weights = (_branch_weights(params, 'dti', ['dti_gat'])
               + _branch_weights(params, 'fmri',
                                 ['fmri_gat_pos', 'fmri_gat_neg']))
    in_specs = [pl.BlockSpec((1, N, N), lambda b: (b, 0, 0)),
                pl.BlockSpec((1, N, N), lambda b: (b, 0, 0))]
    for w in weights:
        in_specs.append(
            pl.BlockSpec(w.shape, functools.partial(
                lambda b, _r: (0,) * _r, _r=w.ndim)))
    out_spec = pl.BlockSpec((1, 1, OUT), lambda b: (b, 0, 0))
    d, f = pl.pallas_call(
        _body,
        grid=(B,),
        in_specs=in_specs,
        out_specs=[out_spec, out_spec],
        out_shape=[jax.ShapeDtypeStruct((B, 1, OUT), _F32)] * 2,
        compiler_params=pltpu.CompilerParams(
            dimension_semantics=("parallel",)),
    )(dti_matrix, fmri_matrix, *weights)
    return d.reshape(B, OUT), f.reshape(B, OUT)


# speedup vs baseline: 1.2335x; 1.0423x over previous
import functools

import jax
import jax.numpy as jnp
from jax.experimental import pallas as pl
from jax.experimental.pallas import tpu as pltpu

N = 400
HID = 64
H = 4
OUT = 32
_F32 = jnp.float32


def _bdot(a, b):
    return jax.lax.dot_general(
        a.astype(jnp.bfloat16), b.astype(jnp.bfloat16),
        (((1,), (0,)), ((), ())), preferred_element_type=_F32)


def _layer_norm_rows(x, g, b, eps=1e-5):
    mu = x.mean(axis=1, keepdims=True)
    var = ((x - mu) ** 2).mean(axis=1, keepdims=True)
    return (x - mu) / jnp.sqrt(var + eps) * g + b


def _gat(x, mask_t, w, att_src, att_dst, bias):
    xh = _bdot(x, w)
    acc = jnp.zeros((N, HID), _F32)
    for h in range(H):
        xh_h = xh[:, h * HID:(h + 1) * HID]
        adst_col = (xh_h * att_dst[h:h + 1, :]).sum(axis=1, keepdims=True)
        asrc_row = jax.lax.dot_general(
            att_src[h:h + 1, :], xh_h, (((1,), (1,)), ((), ())),
            preferred_element_type=_F32)
        e = asrc_row + adst_col
        e = jnp.maximum(e, 0.2 * e)
        e = jnp.where(mask_t, e, -jnp.inf)
        m = jnp.max(e, axis=1, keepdims=True)
        ex = jnp.exp(e - m)
        acc = acc + _bdot(ex, xh_h) / ex.sum(axis=1, keepdims=True)
    return acc * (1.0 / H) + bias


def _pool(v, gate_row):
    s = (v * gate_row).sum(axis=1, keepdims=True)
    p = jnp.exp(s - jnp.max(s))
    p = p / jnp.sum(p)
    return (p * v).sum(axis=0, keepdims=True)


def _branch(mode, m, diag, gmp_w_ref, gmp_b_ref, w1_ref, ln1_g_ref,
            ln1_b_ref, w2_ref, gat_refs, gln_g_ref, gln_b_ref, gate_row_ref,
            ow1_ref, ob1_ref, ow2_ref, ob2_ref, out_ref):
    m3 = m * m * m

    a = jnp.abs(m3)
    rs = a.sum(axis=1, keepdims=True)
    mu_s = jnp.sum(rs) / N
    sd_s = jnp.sqrt(jnp.sum((rs - mu_s) ** 2) / (N - 1)) + 1e-6
    strength = (rs - mu_s) / sd_s
    rsp = rs + 1e-6
    al = (a * jnp.log(a + 1e-9 * rsp)).sum(axis=1, keepdims=True)
    ent = (jnp.log(rsp) * rs - al) / rsp
    mu_e = jnp.sum(ent) / N
    sd_e = jnp.sqrt(jnp.sum((ent - mu_e) ** 2) / (N - 1)) + 1e-6
    ent = (ent - mu_e) / sd_e
    me = (strength * gmp_w_ref[0:1, :] + ent * gmp_w_ref[1:2, :]
          + gmp_b_ref[...])

    aug = 100.0 * m3 + jnp.where(diag, 1.0, 0.0)
    h = _bdot(aug, w1_ref[0:N, :]) + _bdot(me, w1_ref[N:, :])
    h = _layer_norm_rows(h, ln1_g_ref[...], ln1_b_ref[...])
    h = jnp.maximum(h, 0.0)
    h = _bdot(h, w2_ref[...])
    mu_c = h.mean(axis=0, keepdims=True)
    var_c = ((h - mu_c) ** 2).mean(axis=0, keepdims=True)
    x = (h - mu_c) / jnp.sqrt(var_c + 1e-5)

    m3t = m3.T
    if mode == 'dti':
        mask_t = (m3t != 0.0) | diag
        xg = _gat(x, mask_t, *gat_refs[0])
        xg = _layer_norm_rows(xg, gln_g_ref[...], gln_b_ref[...])
    else:
        xp = _gat(x, (m3t > 0.0) | diag, *gat_refs[0])
        xn = _gat(x, (m3t < 0.0) | diag, *gat_refs[1])
        xg = _layer_norm_rows(xp - xn, gln_g_ref[...], gln_b_ref[...])
    xg = jnp.maximum(xg, 0.0) + x

    gate_row = gate_row_ref[...]
    pg = _pool(xg, gate_row)
    pr = _pool(x, gate_row)
    z = (_bdot(pg, ow1_ref[0:HID, :]) + _bdot(pr * 2.0, ow1_ref[HID:, :])
         + ob1_ref[...])
    z = jnp.maximum(z, 0.0)
    z = _bdot(z, ow2_ref[...]) + ob2_ref[...]
    out_ref[...] = z * (1.0 / jnp.sqrt(jnp.float32(1.0 + 1e-5)))


def _unpack(refs, idx, n_gat):
    head = refs[idx:idx + 6]
    idx += 6
    gat_refs = []
    for _ in range(n_gat):
        gat_refs.append(tuple(r[...] for r in refs[idx:idx + 4]))
        idx += 4
    tail = refs[idx:idx + 7]
    return head, gat_refs, tail, idx + 7


def _body(gpp, *refs):
    dti_mat, fmri_mat = refs[0], refs[1]
    d_head, d_gats, d_tail, idx = _unpack(refs, 2, 1)
    f_head, f_gats, f_tail, idx = _unpack(refs, idx, 2)
    out_d, out_f = refs[idx], refs[idx + 1]
    ii = jax.lax.broadcasted_iota(jnp.int32, (N, N), 0)
    jj = jax.lax.broadcasted_iota(jnp.int32, (N, N), 1)
    diag = ii == jj
    for g in range(gpp):
        _branch('dti', dti_mat[g], diag, *d_head, d_gats, *d_tail,
                out_d.at[g])
        _branch('fmri', fmri_mat[g], diag, *f_head, f_gats, *f_tail,
                out_f.at[g])


def _branch_weights(p, pre, gat_names):
    row = lambda v: v.reshape(1, -1)
    ws = [p['gmp_w'], row(p['gmp_b']), p[pre + '_mlp_w1'],
          row(p[pre + '_ln1_g']), row(p[pre + '_ln1_b']), p[pre + '_mlp_w2']]
    for gn in gat_names:
        g = p[gn]
        ws += [g['w'], g['att_src'], g['att_dst'], row(g['bias'])]
    ws += [row(p[pre + '_gln_g']), row(p[pre + '_gln_b']),
           p[pre + '_gate_w'].reshape(1, -1), p[pre + '_out_w1'],
           row(p[pre + '_out_b1']), p[pre + '_out_w2'],
           row(p[pre + '_out_b2'])]
    return ws


def kernel(dti_matrix, fmri_matrix, params):
    B = dti_matrix.shape[0]
    gpp = 2
    weights = (_branch_weights(params, 'dti', ['dti_gat'])
               + _branch_weights(params, 'fmri',
                                 ['fmri_gat_pos', 'fmri_gat_neg']))
    in_specs = [pl.BlockSpec((gpp, N, N), lambda b: (b, 0, 0)),
                pl.BlockSpec((gpp, N, N), lambda b: (b, 0, 0))]
    for w in weights:
        in_specs.append(
            pl.BlockSpec(w.shape, functools.partial(
                lambda b, _r: (0,) * _r, _r=w.ndim)))
    out_spec = pl.BlockSpec((gpp, 1, OUT), lambda b: (b, 0, 0))
    d, f = pl.pallas_call(
        functools.partial(_body, gpp),
        grid=(B // gpp,),
        in_specs=in_specs,
        out_specs=[out_spec, out_spec],
        out_shape=[jax.ShapeDtypeStruct((B, 1, OUT), _F32)] * 2,
        compiler_params=pltpu.CompilerParams(
            dimension_semantics=("parallel",)),
    )(dti_matrix, fmri_matrix, *weights)
    return d.reshape(B, OUT), f.reshape(B, OUT)
